# baseline (speedup 1.0000x reference)
import functools

import jax
import jax.numpy as jnp
from jax import lax
from jax.experimental import pallas as pl
from jax.experimental.pallas import tpu as pltpu
from jax.experimental.pallas import tpu_sc as plsc

NCB, NPOINT, NDIM = 8, 4096, 64
CB = 1024
P = 2048
NPB = NPOINT // P

_NC, _NS = 2, 16
_FB = NDIM // 4
_L = 16


def _vq_body(xt_ref, cbt_ref, dist_ref, oh_ref, idx_ref):
    xt = xt_ref[0]
    cbt = cbt_ref[0]
    cn = jnp.sum(cbt * cbt, axis=0, keepdims=True)
    ones_col = jnp.ones((NDIM, 1), jnp.float32)
    xn = lax.dot_general(xt * xt, ones_col, (((0,), (0,)), ((), ())),
                         preferred_element_type=jnp.float32)
    prod = lax.dot_general(xt, cbt, (((0,), (0,)), ((), ())),
                           preferred_element_type=jnp.float32)
    dist = (xn + cn - 2.0 * prod) * (1.0 / NDIM)

    iota = lax.broadcasted_iota(jnp.int32, (P, CB), 1).astype(jnp.float32)
    m = jnp.min(dist, axis=1, keepdims=True)
    cand = jnp.where(dist == m, iota, float(CB))
    idx = jnp.min(cand, axis=1, keepdims=True)

    dist_ref[0] = dist
    oh_ref[0] = (iota == idx).astype(jnp.float32)
    idx_ref[0] = idx.astype(jnp.int32).reshape(P // _L, _L)


def _vq_tc(x_t, cb_t):
    return pl.pallas_call(
        _vq_body,
        grid=(NCB, NPB),
        in_specs=[
            pl.BlockSpec((1, NDIM, P), lambda g, p: (g, 0, p)),
            pl.BlockSpec((1, NDIM, CB), lambda g, p: (g, 0, 0)),
        ],
        out_specs=[
            pl.BlockSpec((1, P, CB), lambda g, p: (g, p, 0)),
            pl.BlockSpec((1, P, CB), lambda g, p: (g, p, 0)),
            pl.BlockSpec((1, P // _L, _L), lambda g, p: (g, p, 0)),
        ],
        out_shape=[
            jax.ShapeDtypeStruct((NCB, NPOINT, CB), jnp.float32),
            jax.ShapeDtypeStruct((NCB, NPOINT, CB), jnp.float32),
            jax.ShapeDtypeStruct((NCB, NPOINT // _L, _L), jnp.int32),
        ],
        compiler_params=pltpu.CompilerParams(
            dimension_semantics=("parallel", "arbitrary")),
    )(x_t, cb_t)


_sc_mesh = plsc.VectorSubcoreMesh(core_axis_name="c", subcore_axis_name="s")


@functools.partial(
    pl.kernel,
    mesh=_sc_mesh,
    out_type=jax.ShapeDtypeStruct((NCB, NDIM, NPOINT), jnp.float32),
    scratch_types=[
        pltpu.VMEM((_FB, CB), jnp.float32),
        pltpu.VMEM((NPOINT // _L, _L), jnp.int32),
        pltpu.VMEM((_FB, NPOINT), jnp.float32),
    ],
    compiler_params=pltpu.CompilerParams(use_tc_tiling_on_sc=True,
                                         needs_layout_passes=False),
)
def _sc_xhat(cbt_hbm, idx_hbm, out_hbm, cb_v, idx_v, out_v):
    wid = lax.axis_index("s") * _NC + lax.axis_index("c")
    g = wid // 4
    fb = (wid % 4) * _FB
    pltpu.sync_copy(cbt_hbm.at[g, pl.ds(fb, _FB), :], cb_v)
    pltpu.sync_copy(idx_hbm.at[g], idx_v)

    def _feat(f, _):
        rows = jnp.broadcast_to(f, (_L,)).astype(jnp.int32)

        def _chunk(j, _2):
            cols = idx_v[j]
            out_v[f, pl.ds(j * _L, _L)] = plsc.load_gather(cb_v, [rows, cols])
            return 0

        lax.fori_loop(0, NPOINT // _L, _chunk, 0)
        return 0

    lax.fori_loop(0, _FB, _feat, 0)
    pltpu.sync_copy(out_v, out_hbm.at[g, pl.ds(fb, _FB), :])


def kernel(x, code_book):
    x_t = jnp.transpose(x, (0, 2, 1))
    cb_t = jnp.transpose(code_book, (0, 2, 1))
    dist, one_hot, idx = _vq_tc(x_t, cb_t)
    x_hat_t = _sc_xhat(cb_t, idx)
    x_hat = jnp.transpose(x_hat_t, (0, 2, 1))
    return (x_hat, one_hot, dist)


# speedup vs baseline: 1.2115x; 1.2115x over previous
import functools

import jax
import jax.numpy as jnp
from jax import lax
from jax.experimental import pallas as pl
from jax.experimental.pallas import tpu as pltpu
from jax.experimental.pallas import tpu_sc as plsc

NCB, NPOINT, NDIM = 8, 4096, 64
CB = 1024
P = 2048
NPB = NPOINT // P

_NC, _NS = 2, 16
_FB = NDIM // 4
_L = 16


def _vq_body(xt_ref, cbt_ref, dist_ref, oh_ref, idx_ref):
    xt = xt_ref[0]
    cbt = cbt_ref[0]
    cn = jnp.sum(cbt * cbt, axis=0, keepdims=True)
    ones_col = jnp.ones((NDIM, 1), jnp.float32)
    xn = lax.dot_general(xt * xt, ones_col, (((0,), (0,)), ((), ())),
                         preferred_element_type=jnp.float32)
    prod = lax.dot_general(xt, cbt, (((0,), (0,)), ((), ())),
                           preferred_element_type=jnp.float32)
    dist = (xn + cn - 2.0 * prod) * (1.0 / NDIM)

    iota = lax.broadcasted_iota(jnp.int32, (P, CB), 1).astype(jnp.float32)
    m = jnp.min(dist, axis=1, keepdims=True)
    cand = jnp.where(dist == m, iota, float(CB))
    idx = jnp.min(cand, axis=1, keepdims=True)

    dist_ref[0] = dist
    oh_ref[0] = (iota == idx).astype(jnp.float32)
    idx_ref[0] = idx.astype(jnp.int32).reshape(P // _L, _L)


def _vq_tc(x_t, cb_t):
    return pl.pallas_call(
        _vq_body,
        grid=(NCB, NPB),
        in_specs=[
            pl.BlockSpec((1, NDIM, P), lambda g, p: (g, 0, p)),
            pl.BlockSpec((1, NDIM, CB), lambda g, p: (g, 0, 0)),
        ],
        out_specs=[
            pl.BlockSpec((1, P, CB), lambda g, p: (g, p, 0)),
            pl.BlockSpec((1, P, CB), lambda g, p: (g, p, 0)),
            pl.BlockSpec((1, P // _L, _L), lambda g, p: (g, p, 0)),
        ],
        out_shape=[
            jax.ShapeDtypeStruct((NCB, NPOINT, CB), jnp.float32),
            jax.ShapeDtypeStruct((NCB, NPOINT, CB), jnp.float32),
            jax.ShapeDtypeStruct((NCB, NPOINT // _L, _L), jnp.int32),
        ],
        compiler_params=pltpu.CompilerParams(
            dimension_semantics=("parallel", "arbitrary")),
    )(x_t, cb_t)


_sc_mesh = plsc.VectorSubcoreMesh(core_axis_name="c", subcore_axis_name="s")


@functools.partial(
    pl.kernel,
    mesh=_sc_mesh,
    out_type=jax.ShapeDtypeStruct((NCB, NDIM, NPOINT), jnp.float32),
    scratch_types=[
        pltpu.VMEM((_FB, CB), jnp.float32),
        pltpu.VMEM((NPOINT // _L, _L), jnp.int32),
        pltpu.VMEM((_FB, NPOINT), jnp.float32),
    ],
    compiler_params=pltpu.CompilerParams(use_tc_tiling_on_sc=True,
                                         needs_layout_passes=False),
)
def _sc_xhat(cbt_hbm, idx_hbm, out_hbm, cb_v, idx_v, out_v):
    wid = lax.axis_index("s") * _NC + lax.axis_index("c")
    g = wid // 4
    fb = (wid % 4) * _FB
    pltpu.sync_copy(cbt_hbm.at[g, pl.ds(fb, _FB), :], cb_v)
    pltpu.sync_copy(idx_hbm.at[g], idx_v)

    def _chunk(j, _):
        cols = idx_v[j]
        for f in range(_FB):
            rows = jnp.full((_L,), f, jnp.int32)
            out_v[f, pl.ds(j * _L, _L)] = plsc.load_gather(cb_v, [rows, cols])
        return 0

    lax.fori_loop(0, NPOINT // _L, _chunk, 0)
    pltpu.sync_copy(out_v, out_hbm.at[g, pl.ds(fb, _FB), :])


def kernel(x, code_book):
    x_t = jnp.transpose(x, (0, 2, 1))
    cb_t = jnp.transpose(code_book, (0, 2, 1))
    dist, one_hot, idx = _vq_tc(x_t, cb_t)
    x_hat_t = _sc_xhat(cb_t, idx)
    x_hat = jnp.transpose(x_hat_t, (0, 2, 1))
    return (x_hat, one_hot, dist)


# speedup vs baseline: 1.2190x; 1.0062x over previous
import functools

import jax
import jax.numpy as jnp
from jax import lax
from jax.experimental import pallas as pl
from jax.experimental.pallas import tpu as pltpu
from jax.experimental.pallas import tpu_sc as plsc

NCB, NPOINT, NDIM = 8, 4096, 64
CB = 1024
P = 2048
NPB = NPOINT // P

_NC, _NS = 2, 16
_FB = NDIM // 4
_L = 16


def _vq_body(xt_ref, cbt_ref, dist_ref, oh_ref, idx_ref):
    xt = xt_ref[0]
    cbt = cbt_ref[0]
    cn = jnp.sum(cbt * cbt, axis=0, keepdims=True)
    ones_col = jnp.ones((NDIM, 1), jnp.float32)
    xn = lax.dot_general(xt * xt, ones_col, (((0,), (0,)), ((), ())),
                         preferred_element_type=jnp.float32)
    prod = lax.dot_general(xt, cbt, (((0,), (0,)), ((), ())),
                           preferred_element_type=jnp.float32)
    dist = (xn + cn - 2.0 * prod) * (1.0 / NDIM)

    iota = lax.broadcasted_iota(jnp.int32, (P, CB), 1).astype(jnp.float32)
    m = jnp.min(dist, axis=1, keepdims=True)
    cand = jnp.where(dist == m, iota, float(CB))
    idx = jnp.min(cand, axis=1, keepdims=True)

    dist_ref[0] = dist
    oh_ref[0] = (iota == idx).astype(jnp.float32)
    idx_ref[0] = idx.astype(jnp.int32).reshape(P // _L, _L)


def _vq_tc(x_t, cb_t):
    return pl.pallas_call(
        _vq_body,
        grid=(NCB, NPB),
        in_specs=[
            pl.BlockSpec((1, NDIM, P), lambda g, p: (g, 0, p)),
            pl.BlockSpec((1, NDIM, CB), lambda g, p: (g, 0, 0)),
        ],
        out_specs=[
            pl.BlockSpec((1, P, CB), lambda g, p: (g, p, 0)),
            pl.BlockSpec((1, P, CB), lambda g, p: (g, p, 0)),
            pl.BlockSpec((1, P // _L, _L), lambda g, p: (g, p, 0)),
        ],
        out_shape=[
            jax.ShapeDtypeStruct((NCB, NPOINT, CB), jnp.float32),
            jax.ShapeDtypeStruct((NCB, NPOINT, CB), jnp.float32),
            jax.ShapeDtypeStruct((NCB, NPOINT // _L, _L), jnp.int32),
        ],
        compiler_params=pltpu.CompilerParams(
            dimension_semantics=("parallel", "arbitrary")),
    )(x_t, cb_t)


_sc_mesh = plsc.VectorSubcoreMesh(core_axis_name="c", subcore_axis_name="s")


@functools.partial(
    pl.kernel,
    mesh=_sc_mesh,
    out_type=jax.ShapeDtypeStruct((NCB, NDIM, NPOINT), jnp.float32),
    scratch_types=[
        pltpu.VMEM((_FB, CB), jnp.float32),
        pltpu.VMEM((NPOINT // _L, _L), jnp.int32),
        pltpu.VMEM((_FB, NPOINT), jnp.float32),
    ],
    compiler_params=pltpu.CompilerParams(use_tc_tiling_on_sc=True,
                                         needs_layout_passes=False),
)
def _sc_xhat(cbt_hbm, idx_hbm, out_hbm, cb_v, idx_v, out_v):
    wid = lax.axis_index("s") * _NC + lax.axis_index("c")
    g = wid // 4
    fb = (wid % 4) * _FB
    pltpu.sync_copy(cbt_hbm.at[g, pl.ds(fb, _FB), :], cb_v)
    pltpu.sync_copy(idx_hbm.at[g], idx_v)

    def _chunk(j2, _):
        for u in range(2):
            j = j2 * 2 + u
            cols = idx_v[j]
            for f in range(_FB):
                rows = jnp.full((_L,), f, jnp.int32)
                out_v[f, pl.ds(j * _L, _L)] = plsc.load_gather(cb_v,
                                                               [rows, cols])
        return 0

    lax.fori_loop(0, NPOINT // _L // 2, _chunk, 0)
    pltpu.sync_copy(out_v, out_hbm.at[g, pl.ds(fb, _FB), :])


def kernel(x, code_book):
    x_t = jnp.transpose(x, (0, 2, 1))
    cb_t = jnp.transpose(code_book, (0, 2, 1))
    dist, one_hot, idx = _vq_tc(x_t, cb_t)
    x_hat_t = _sc_xhat(cb_t, idx)
    x_hat = jnp.transpose(x_hat_t, (0, 2, 1))
    return (x_hat, one_hot, dist)
